# Initial kernel scaffold; baseline (speedup 1.0000x reference)
import jax
import jax.numpy as jnp
from jax.experimental import pallas as pl


def kernel(x, router_enc_w, router_enc_b, router_dec_w, decoder_bias, down_w, up_w, child_enc_w, child_enc_b, child_dec_w):
    raise NotImplementedError("write your pallas kernel here")


# speedup vs baseline: 3.0719x; 3.0719x over previous
import jax
import jax.numpy as jnp
from jax.experimental import pallas as pl
from jax.experimental.pallas import tpu as pltpu

D = 2048
P = 16
SUB = 256
C = 64
PC = P * C
BETA = 0.1
NEG = -3.4e38
BT = 512


def _fold_kernel(uw_ref, cdw_ref, rd_ref, bias_ref, tr_ref):
    p = pl.program_id(0)
    uw = uw_ref[0]
    cdw = cdw_ref[0]
    t = jax.lax.dot_general(
        cdw, uw, (((0,), (1,)), ((), ())), preferred_element_type=jnp.float32)
    sel = (jax.lax.broadcasted_iota(jnp.int32, (P, 1), 0) == p).astype(jnp.float32)
    rd_row = jax.lax.dot_general(
        sel, rd_ref[...], (((0,), (1,)), ((), ())), preferred_element_type=jnp.float32)
    tr_ref[...] = t + BETA * rd_row + 0.5 * bias_ref[...]


def _main_kernel(x_ref, rew_ref, reb_ref, ceb_ref, dw_ref, cew_ref, tr_ref,
                 bias_ref, out_ref, xb_ref, i1_ref, i2_ref, c1_ref, c2_ref):
    p = pl.program_id(1)

    @pl.when(p == 0)
    def _():
        xb = x_ref[...] - bias_ref[...]
        xb_ref[...] = xb
        plog = jax.lax.dot_general(
            xb, rew_ref[...], (((1,), (1,)), ((), ())),
            preferred_element_type=jnp.float32) + reb_ref[...]
        iota_p = jax.lax.broadcasted_iota(jnp.int32, plog.shape, 1)
        m1 = jnp.max(plog, axis=1, keepdims=True)
        i1 = jnp.min(jnp.where(plog == m1, iota_p, P), axis=1, keepdims=True)
        plog2 = jnp.where(iota_p == i1, NEG, plog)
        m2 = jnp.max(plog2, axis=1, keepdims=True)
        i2 = jnp.min(jnp.where(plog2 == m2, iota_p, P), axis=1, keepdims=True)
        i1_ref[...] = i1
        i2_ref[...] = i2

    xb = xb_ref[...]
    z = jax.lax.dot_general(
        xb, dw_ref[0], (((1,), (1,)), ((), ())),
        preferred_element_type=jnp.float32)
    clog = jax.lax.dot_general(
        z, cew_ref[0], (((1,), (1,)), ((), ())),
        preferred_element_type=jnp.float32) + ceb_ref[0]
    iota_c = jax.lax.broadcasted_iota(jnp.int32, clog.shape, 1)
    mx = jnp.max(clog, axis=1, keepdims=True)
    cid = jnp.min(jnp.where(clog == mx, iota_c, C), axis=1, keepdims=True)

    i1 = i1_ref[...]
    i2 = i2_ref[...]
    c1 = jnp.where(i1 == p, cid, c1_ref[...])
    c2 = jnp.where(i2 == p, cid, c2_ref[...])
    c1_ref[...] = c1
    c2_ref[...] = c2

    @pl.when(p == P - 1)
    def _():
        f1 = i1 * C + c1
        f2 = i2 * C + c2
        iota_pc = jax.lax.broadcasted_iota(jnp.int32, (xb.shape[0], PC), 1)
        onehot = ((iota_pc == f1) | (iota_pc == f2)).astype(jnp.float32)
        out_ref[...] = jax.lax.dot_general(
            onehot, tr_ref[...], (((1,), (0,)), ((), ())),
            preferred_element_type=jnp.float32)


def kernel(x, router_enc_w, router_enc_b, router_dec_w, decoder_bias,
           down_w, up_w, child_enc_w, child_enc_b, child_dec_w):
    B = x.shape[0]
    bias_row = decoder_bias.reshape(1, D)
    reb = router_enc_b.reshape(1, P)

    trows = pl.pallas_call(
        _fold_kernel,
        grid=(P,),
        in_specs=[
            pl.BlockSpec((1, D, SUB), lambda p: (p, 0, 0)),
            pl.BlockSpec((1, SUB, C), lambda p: (p, 0, 0)),
            pl.BlockSpec((D, P), lambda p: (0, 0)),
            pl.BlockSpec((1, D), lambda p: (0, 0)),
        ],
        out_specs=pl.BlockSpec((C, D), lambda p: (p, 0)),
        out_shape=jax.ShapeDtypeStruct((PC, D), jnp.float32),
        compiler_params=pltpu.CompilerParams(
            dimension_semantics=("parallel",)),
    )(up_w, child_dec_w, router_dec_w, bias_row)

    out = pl.pallas_call(
        _main_kernel,
        grid=(B // BT, P),
        in_specs=[
            pl.BlockSpec((BT, D), lambda i, p: (i, 0)),
            pl.BlockSpec((P, D), lambda i, p: (0, 0)),
            pl.BlockSpec((1, P), lambda i, p: (0, 0)),
            pl.BlockSpec((1, 1, C), lambda i, p: (p, 0, 0)),
            pl.BlockSpec((1, SUB, D), lambda i, p: (p, 0, 0)),
            pl.BlockSpec((1, C, SUB), lambda i, p: (p, 0, 0)),
            pl.BlockSpec((PC, D), lambda i, p: (0, 0)),
            pl.BlockSpec((1, D), lambda i, p: (0, 0)),
        ],
        out_specs=pl.BlockSpec((BT, D), lambda i, p: (i, 0)),
        out_shape=jax.ShapeDtypeStruct((B, D), jnp.float32),
        scratch_shapes=[
            pltpu.VMEM((BT, D), jnp.float32),
            pltpu.VMEM((BT, 1), jnp.int32),
            pltpu.VMEM((BT, 1), jnp.int32),
            pltpu.VMEM((BT, 1), jnp.int32),
            pltpu.VMEM((BT, 1), jnp.int32),
        ],
        compiler_params=pltpu.CompilerParams(
            dimension_semantics=("arbitrary", "arbitrary")),
    )(x, router_enc_w, reb, child_enc_b.reshape(P, 1, C), down_w, child_enc_w,
      trows, bias_row)
    return out


# speedup vs baseline: 5.0605x; 1.6473x over previous
import jax
import jax.numpy as jnp
from jax.experimental import pallas as pl
from jax.experimental.pallas import tpu as pltpu

D = 2048
P = 16
SUB = 256
C = 64
PC = P * C
BETA = 0.1
NEG = -3.4e38
BT = 512
BTC = 2048


def _fold_kernel(uw_ref, cdw_ref, rd_ref, bias_ref, tr_ref):
    p = pl.program_id(0)
    uw = uw_ref[0]
    cdw = cdw_ref[0]
    t = jax.lax.dot_general(
        cdw, uw, (((0,), (1,)), ((), ())), preferred_element_type=jnp.float32)
    sel = (jax.lax.broadcasted_iota(jnp.int32, (P, 1), 0) == p).astype(jnp.float32)
    rd_row = jax.lax.dot_general(
        sel, rd_ref[...], (((0,), (1,)), ((), ())), preferred_element_type=jnp.float32)
    tr_ref[...] = t + BETA * rd_row + 0.5 * bias_ref[...]


def _select_kernel(x_ref, rew_ref, reb_ref, ceb_ref, dw_ref, cew_ref,
                   bias_ref, f1_ref, f2_ref):
    xb = x_ref[...] - bias_ref[...]
    plog = jax.lax.dot_general(
        xb, rew_ref[...], (((1,), (1,)), ((), ())),
        preferred_element_type=jnp.float32) + reb_ref[...]
    iota_p = jax.lax.broadcasted_iota(jnp.int32, plog.shape, 1)
    m1 = jnp.max(plog, axis=1, keepdims=True)
    i1 = jnp.min(jnp.where(plog == m1, iota_p, P), axis=1, keepdims=True)
    plog2 = jnp.where(iota_p == i1, NEG, plog)
    m2 = jnp.max(plog2, axis=1, keepdims=True)
    i2 = jnp.min(jnp.where(plog2 == m2, iota_p, P), axis=1, keepdims=True)

    z = jax.lax.dot_general(
        xb, dw_ref[...], (((1,), (1,)), ((), ())),
        preferred_element_type=jnp.float32)
    c1 = jnp.zeros_like(i1)
    c2 = jnp.zeros_like(i2)
    iota_c = jax.lax.broadcasted_iota(jnp.int32, (z.shape[0], C), 1)
    for p in range(P):
        clog = jax.lax.dot_general(
            z[:, p * SUB:(p + 1) * SUB], cew_ref[p],
            (((1,), (1,)), ((), ())),
            preferred_element_type=jnp.float32) + ceb_ref[:, p * C:(p + 1) * C]
        mx = jnp.max(clog, axis=1, keepdims=True)
        cid = jnp.min(jnp.where(clog == mx, iota_c, C), axis=1, keepdims=True)
        c1 = jnp.where(i1 == p, cid, c1)
        c2 = jnp.where(i2 == p, cid, c2)
    f1_ref[...] = i1 * C + c1
    f2_ref[...] = i2 * C + c2


def _combine_kernel(f1_ref, f2_ref, tr_ref, out_ref):
    f1 = f1_ref[...]
    f2 = f2_ref[...]
    iota_pc = jax.lax.broadcasted_iota(jnp.int32, (f1.shape[0], PC), 1)
    onehot = ((iota_pc == f1) | (iota_pc == f2)).astype(jnp.float32)
    out_ref[...] = jax.lax.dot_general(
        onehot, tr_ref[...], (((1,), (0,)), ((), ())),
        preferred_element_type=jnp.float32)


def kernel(x, router_enc_w, router_enc_b, router_dec_w, decoder_bias,
           down_w, up_w, child_enc_w, child_enc_b, child_dec_w):
    B = x.shape[0]
    bias_row = decoder_bias.reshape(1, D)
    reb = router_enc_b.reshape(1, P)
    ceb = child_enc_b.reshape(1, PC)
    dw_flat = down_w.reshape(P * SUB, D)

    trows = pl.pallas_call(
        _fold_kernel,
        grid=(P,),
        in_specs=[
            pl.BlockSpec((1, D, SUB), lambda p: (p, 0, 0)),
            pl.BlockSpec((1, SUB, C), lambda p: (p, 0, 0)),
            pl.BlockSpec((D, P), lambda p: (0, 0)),
            pl.BlockSpec((1, D), lambda p: (0, 0)),
        ],
        out_specs=pl.BlockSpec((C, D), lambda p: (p, 0)),
        out_shape=jax.ShapeDtypeStruct((PC, D), jnp.float32),
        compiler_params=pltpu.CompilerParams(
            dimension_semantics=("parallel",)),
    )(up_w, child_dec_w, router_dec_w, bias_row)

    f1, f2 = pl.pallas_call(
        _select_kernel,
        grid=(B // BT,),
        in_specs=[
            pl.BlockSpec((BT, D), lambda i: (i, 0)),
            pl.BlockSpec((P, D), lambda i: (0, 0)),
            pl.BlockSpec((1, P), lambda i: (0, 0)),
            pl.BlockSpec((1, PC), lambda i: (0, 0)),
            pl.BlockSpec((P * SUB, D), lambda i: (0, 0)),
            pl.BlockSpec((P, C, SUB), lambda i: (0, 0, 0)),
            pl.BlockSpec((1, D), lambda i: (0, 0)),
        ],
        out_specs=[
            pl.BlockSpec((BT, 1), lambda i: (i, 0)),
            pl.BlockSpec((BT, 1), lambda i: (i, 0)),
        ],
        out_shape=[
            jax.ShapeDtypeStruct((B, 1), jnp.int32),
            jax.ShapeDtypeStruct((B, 1), jnp.int32),
        ],
        compiler_params=pltpu.CompilerParams(
            dimension_semantics=("arbitrary",)),
    )(x, router_enc_w, reb, ceb, dw_flat, child_enc_w, bias_row)

    out = pl.pallas_call(
        _combine_kernel,
        grid=(B // BTC,),
        in_specs=[
            pl.BlockSpec((BTC, 1), lambda i: (i, 0)),
            pl.BlockSpec((BTC, 1), lambda i: (i, 0)),
            pl.BlockSpec((PC, D), lambda i: (0, 0)),
        ],
        out_specs=pl.BlockSpec((BTC, D), lambda i: (i, 0)),
        out_shape=jax.ShapeDtypeStruct((B, D), jnp.float32),
        compiler_params=pltpu.CompilerParams(
            dimension_semantics=("arbitrary",)),
    )(f1, f2, trows)
    return out
